# Initial kernel scaffold; baseline (speedup 1.0000x reference)
import functools

import jax
import jax.numpy as jnp
from jax import lax
from jax.experimental import pallas as pl
from jax.experimental.pallas import tpu as pltpu
from jax.experimental.pallas import tpu_sc as plsc

N_NODES = 10000
N_EDGES = 320000
D = 128
DP = 144
NEG_SLOPE = 0.2

NC = 2
NS = 16
NW = NC * NS
CHUNK = 80
CHUNKS_PER_TILE = N_EDGES // (NW * CHUNK)
ROWS_PER_TILE = N_NODES // NS
ZROWS = 125


def _tc_pre_body(x_ref, w_ref, ap_ref, msg_ref, al_ref):
    msg = jnp.dot(x_ref[...], w_ref[...], preferred_element_type=jnp.float32)
    msg_ref[...] = msg
    al_ref[...] = jnp.dot(msg, ap_ref[...], preferred_element_type=jnp.float32)


def _tc_pre(x, w, a_pad):
    blk = 1000
    grid = N_NODES // blk
    return pl.pallas_call(
        _tc_pre_body,
        grid=(grid,),
        in_specs=[
            pl.BlockSpec((blk, D), lambda i: (i, 0)),
            pl.BlockSpec((D, D), lambda i: (0, 0)),
            pl.BlockSpec((D, D), lambda i: (0, 0)),
        ],
        out_specs=[
            pl.BlockSpec((blk, D), lambda i: (i, 0)),
            pl.BlockSpec((blk, D), lambda i: (i, 0)),
        ],
        out_shape=[
            jax.ShapeDtypeStruct((N_NODES, D), jnp.float32),
            jax.ShapeDtypeStruct((N_NODES, D), jnp.float32),
        ],
    )(x, w, a_pad)


def _sc_body(row_h, col_h, nv_h, asrc_h, adst_h, msg_h, num_h,
             row_v, col_v, nv_v, asrc_v, adst_v, rows_v, sc_rows_v,
             w_v, ex_v, zbuf_v, acc_sh, sem):
    cid = lax.axis_index("c")
    sid = lax.axis_index("s")
    wid = cid * NS + sid

    pltpu.sync_copy(row_h.at[pl.ds(wid * CHUNKS_PER_TILE, CHUNKS_PER_TILE)], row_v)
    pltpu.sync_copy(col_h.at[pl.ds(wid * CHUNKS_PER_TILE, CHUNKS_PER_TILE)], col_v)
    pltpu.sync_copy(nv_h.at[pl.ds(wid * CHUNKS_PER_TILE, CHUNKS_PER_TILE)], nv_v)
    pltpu.sync_copy(asrc_h, asrc_v)
    pltpu.sync_copy(adst_h, adst_v)

    z16 = jnp.zeros((16,), jnp.float32)

    def zero_row(r, _):
        for k in range(DP // 16):
            zbuf_v[r, pl.ds(k * 16, 16)] = z16
        return 0

    lax.fori_loop(0, ZROWS, zero_row, 0)
    for k in range(ROWS_PER_TILE // ZROWS):
        pltpu.sync_copy(zbuf_v, acc_sh.at[pl.ds(sid * ROWS_PER_TILE + k * ZROWS, ZROWS)])
    plsc.subcore_barrier()

    lane0 = lax.broadcasted_iota(jnp.int32, (16,), 0) == 0

    def chunk(j, _):
        pltpu.async_copy(msg_h.at[col_v.at[j]], rows_v, sem).wait()

        for g in range(CHUNK // 16):
            r16 = row_v[j, pl.ds(g * 16, 16)]
            c16 = col_v[j, pl.ds(g * 16, 16)]
            logit = (plsc.load_gather(asrc_v, [r16]) +
                     plsc.load_gather(adst_v, [c16]))
            logit = jnp.where(logit >= 0, logit, NEG_SLOPE * logit)
            ex = jnp.exp(logit)
            w_v[pl.ds(g * 16, 16)] = nv_v[j, pl.ds(g * 16, 16)] * ex
            ex_v[pl.ds(g * 16, 16)] = ex

        def edge(i, _):
            w = w_v[i]
            for k in range(D // 16):
                sc_rows_v[i, pl.ds(k * 16, 16)] = rows_v[i, pl.ds(k * 16, 16)] * w
            sc_rows_v[i, pl.ds(D, 16)] = jnp.where(lane0, ex_v[i], 0.0)
            return 0

        lax.fori_loop(0, CHUNK, edge, 0)

        pltpu.sync_copy(sc_rows_v, acc_sh.at[row_v.at[j]], add=True)
        return 0

    lax.fori_loop(0, CHUNKS_PER_TILE, chunk, 0)

    plsc.subcore_barrier()
    for k in range(ROWS_PER_TILE // ZROWS):
        base = sid * ROWS_PER_TILE + k * ZROWS
        pltpu.sync_copy(acc_sh.at[pl.ds(base, ZROWS)],
                        num_h.at[cid, pl.ds(base, ZROWS)])


def _sc_pass(row2d, col2d, nv2d, a_src, a_dst, msg):
    mesh = plsc.VectorSubcoreMesh(core_axis_name="c", subcore_axis_name="s")
    f = pl.kernel(
        _sc_body,
        out_type=jax.ShapeDtypeStruct((NC, N_NODES, DP), jnp.float32),
        mesh=mesh,
        scratch_types=[
            pltpu.VMEM((CHUNKS_PER_TILE, CHUNK), jnp.int32),
            pltpu.VMEM((CHUNKS_PER_TILE, CHUNK), jnp.int32),
            pltpu.VMEM((CHUNKS_PER_TILE, CHUNK), jnp.float32),
            pltpu.VMEM((N_NODES,), jnp.float32),
            pltpu.VMEM((N_NODES,), jnp.float32),
            pltpu.VMEM((CHUNK, D), jnp.float32),
            pltpu.VMEM((CHUNK, DP), jnp.float32),
            pltpu.VMEM((CHUNK,), jnp.float32),
            pltpu.VMEM((CHUNK,), jnp.float32),
            pltpu.VMEM((ZROWS, DP), jnp.float32),
            pltpu.VMEM_SHARED((N_NODES, DP), jnp.float32),
            pltpu.SemaphoreType.DMA,
        ],
    )
    return f(row2d, col2d, nv2d, a_src, a_dst, msg)


def _tc_fin_body(np_ref, out_ref):
    n0 = np_ref[0]
    n1 = np_ref[1]
    num = n0[:, :D] + n1[:, :D]
    den = n0[:, D:D + 1] + n1[:, D:D + 1]
    safe = den > 0
    inv = jnp.where(safe, 1.0 / jnp.where(safe, den, 1.0), 0.0)
    out_ref[...] = num * inv


def _tc_fin(num_part):
    blk = 1000
    grid = N_NODES // blk
    return pl.pallas_call(
        _tc_fin_body,
        grid=(grid,),
        in_specs=[pl.BlockSpec((NC, blk, DP), lambda i: (0, i, 0))],
        out_specs=pl.BlockSpec((blk, D), lambda i: (i, 0)),
        out_shape=jax.ShapeDtypeStruct((N_NODES, D), jnp.float32),
    )(num_part)


@jax.jit
def kernel(x, edge_index, neighborhood_values, W, a):
    row2d = edge_index[0].reshape(NW * CHUNKS_PER_TILE, CHUNK)
    col2d = edge_index[1].reshape(NW * CHUNKS_PER_TILE, CHUNK)
    nv2d = neighborhood_values.reshape(NW * CHUNKS_PER_TILE, CHUNK)
    a_pad = jnp.zeros((D, D), jnp.float32)
    a_pad = a_pad.at[:, 0].set(a[:D, 0]).at[:, 1].set(a[D:, 0])

    msg, alphas = _tc_pre(x, W, a_pad)
    a_src = alphas[:, 0]
    a_dst = alphas[:, 1]

    num_part = _sc_pass(row2d, col2d, nv2d, a_src, a_dst, msg)
    return _tc_fin(num_part)


# speedup vs baseline: 16.6903x; 16.6903x over previous
import jax
import jax.numpy as jnp
from jax import lax
from jax.experimental import pallas as pl
from jax.experimental.pallas import tpu as pltpu
from jax.experimental.pallas import tpu_sc as plsc

N_NODES = 10000
N_EDGES = 320000
D = 128
NEG_SLOPE = 0.2

NC = 2
NS = 16
NW = NC * NS
CHUNK = 80
CHUNKS_PER_TILE = N_EDGES // (NW * CHUNK)
NSTAGE = 25
ECB = CHUNKS_PER_TILE // NSTAGE
ACH = N_NODES // CHUNK


def _tc_pre_body(x_ref, w_ref, ap_ref, msg_ref, al_ref):
    msg = jnp.dot(x_ref[...], w_ref[...], preferred_element_type=jnp.float32)
    msg_ref[...] = msg
    al_ref[...] = jnp.dot(msg, ap_ref[...], preferred_element_type=jnp.float32)


def _tc_pre(x, w, a_pad):
    blk = 1000
    grid = N_NODES // blk
    return pl.pallas_call(
        _tc_pre_body,
        grid=(grid,),
        in_specs=[
            pl.BlockSpec((blk, D), lambda i: (i, 0)),
            pl.BlockSpec((D, D), lambda i: (0, 0)),
            pl.BlockSpec((D, D), lambda i: (0, 0)),
        ],
        out_specs=[
            pl.BlockSpec((blk, D), lambda i: (i, 0)),
            pl.BlockSpec((blk, D), lambda i: (i, 0)),
        ],
        out_shape=[
            jax.ShapeDtypeStruct((N_NODES, D), jnp.float32),
            jax.ShapeDtypeStruct((N_NODES, D), jnp.float32),
        ],
    )(x, w, a_pad)


def _sc_body(row_h, col_h, nv_h, asrc_h, adst_h, msg_h, num_h, den_h,
             row_v, col_v, nv_v, asrc_v, adst_v, rows_v,
             w_v, den_v, acc_sh, sem):
    cid = lax.axis_index("c")
    sid = lax.axis_index("s")
    wid = cid * NS + sid

    pltpu.sync_copy(asrc_h, asrc_v)
    pltpu.sync_copy(adst_h, adst_v)

    z16 = jnp.zeros((16,), jnp.float32)

    def zero_den(r, _):
        den_v[pl.ds(r * 16, 16)] = z16
        return 0

    lax.fori_loop(0, N_NODES // 16, zero_den, 0)

    def zero_row(r, _):
        for k in range(D // 16):
            rows_v[r, pl.ds(k * 16, 16)] = z16
        return 0

    lax.fori_loop(0, CHUNK, zero_row, 0)
    for k in range(ACH // NS + 1):
        t = sid + NS * k

        @pl.when(t < ACH)
        def _():
            pltpu.sync_copy(rows_v, acc_sh.at[pl.ds(t * CHUNK, CHUNK)])

    plsc.subcore_barrier()

    def stage(s, _):
        pltpu.sync_copy(row_h.at[wid, s], row_v)
        pltpu.sync_copy(col_h.at[wid, s], col_v)
        pltpu.sync_copy(nv_h.at[wid, s], nv_v)

        def chunk(j, _):
            pltpu.async_copy(msg_h.at[col_v.at[j]], rows_v, sem).wait()

            for g in range(CHUNK // 16):
                r16 = row_v[j, pl.ds(g * 16, 16)]
                c16 = col_v[j, pl.ds(g * 16, 16)]
                logit = (plsc.load_gather(asrc_v, [r16]) +
                         plsc.load_gather(adst_v, [c16]))
                logit = jnp.where(logit >= 0, logit, NEG_SLOPE * logit)
                ex = jnp.exp(logit)
                w_v[pl.ds(g * 16, 16)] = nv_v[j, pl.ds(g * 16, 16)] * ex
                plsc.addupdate_scatter(den_v, [r16], ex)

            def edge(i, _):
                idx = jnp.full((16,), i, jnp.int32)
                w = plsc.load_gather(w_v, [idx])
                for k in range(D // 16):
                    rows_v[i, pl.ds(k * 16, 16)] = rows_v[i, pl.ds(k * 16, 16)] * w
                return 0

            lax.fori_loop(0, CHUNK, edge, 0)

            pltpu.sync_copy(rows_v, acc_sh.at[row_v.at[j]], add=True)
            return 0

        lax.fori_loop(0, ECB, chunk, 0)
        return 0

    lax.fori_loop(0, NSTAGE, stage, 0)

    pltpu.sync_copy(den_v, den_h.at[cid, sid, 0])

    plsc.subcore_barrier()
    for k in range(ACH // NS + 1):
        t = sid + NS * k

        @pl.when(t < ACH)
        def _():
            pltpu.sync_copy(acc_sh.at[pl.ds(t * CHUNK, CHUNK)],
                            num_h.at[cid, pl.ds(t * CHUNK, CHUNK)])


def _sc_pass(row3d, col3d, nv3d, a_src, a_dst, msg):
    mesh = plsc.VectorSubcoreMesh(core_axis_name="c", subcore_axis_name="s")
    f = pl.kernel(
        _sc_body,
        out_type=[
            jax.ShapeDtypeStruct((NC, N_NODES, D), jnp.float32),
            jax.ShapeDtypeStruct((NC, NS, 1, N_NODES), jnp.float32),
        ],
        mesh=mesh,
        scratch_types=[
            pltpu.VMEM((ECB, CHUNK), jnp.int32),
            pltpu.VMEM((ECB, CHUNK), jnp.int32),
            pltpu.VMEM((ECB, CHUNK), jnp.float32),
            pltpu.VMEM((N_NODES,), jnp.float32),
            pltpu.VMEM((N_NODES,), jnp.float32),
            pltpu.VMEM((CHUNK, D), jnp.float32),
            pltpu.VMEM((CHUNK,), jnp.float32),
            pltpu.VMEM((N_NODES,), jnp.float32),
            pltpu.VMEM_SHARED((N_NODES, D), jnp.float32),
            pltpu.SemaphoreType.DMA,
        ],
        compiler_params=pltpu.CompilerParams(needs_layout_passes=False),
    )
    return f(row3d, col3d, nv3d, a_src, a_dst, msg)


def _tc_fin_body(np_ref, dp_ref, out_ref):
    n0 = np_ref[0]
    n1 = np_ref[1]
    num = n0 + n1
    den = jnp.sum(dp_ref[...], axis=1)[:, None]
    safe = den > 0
    inv = jnp.where(safe, 1.0 / jnp.where(safe, den, 1.0), 0.0)
    out_ref[...] = num * inv


def _tc_fin(num_part, den_part):
    blk = 1000
    grid = N_NODES // blk
    return pl.pallas_call(
        _tc_fin_body,
        grid=(grid,),
        in_specs=[
            pl.BlockSpec((NC, blk, D), lambda i: (0, i, 0)),
            pl.BlockSpec((blk, NW), lambda i: (i, 0)),
        ],
        out_specs=pl.BlockSpec((blk, D), lambda i: (i, 0)),
        out_shape=jax.ShapeDtypeStruct((N_NODES, D), jnp.float32),
    )(num_part, den_part)


@jax.jit
def kernel(x, edge_index, neighborhood_values, W, a):
    row3d = edge_index[0].reshape(NW, NSTAGE, ECB, CHUNK)
    col3d = edge_index[1].reshape(NW, NSTAGE, ECB, CHUNK)
    nv3d = neighborhood_values.reshape(NW, NSTAGE, ECB, CHUNK)
    a_pad = jnp.zeros((D, D), jnp.float32)
    a_pad = a_pad.at[:, 0].set(a[:D, 0]).at[:, 1].set(a[D:, 0])

    msg, alphas = _tc_pre(x, W, a_pad)
    a_src = alphas[:, 0]
    a_dst = alphas[:, 1]

    num_part, den_part = _sc_pass(row3d, col3d, nv3d, a_src, a_dst, msg)
    den2d = den_part.reshape(NW, N_NODES).T
    return _tc_fin(num_part, den2d)


# speedup vs baseline: 28.8705x; 1.7298x over previous
import jax
import jax.numpy as jnp
from jax import lax
from jax.experimental import pallas as pl
from jax.experimental.pallas import tpu as pltpu
from jax.experimental.pallas import tpu_sc as plsc

N_NODES = 10000
N_EDGES = 320000
D = 128
NEG_SLOPE = 0.2

NC = 2
NS = 16
NW = NC * NS
CHUNK = 80
NCH = N_EDGES // (NW * CHUNK)
NSA = 5
ECA = NCH // NSA
SBB = 5
NST = NCH // SBB
SUP = 15
NOUT = (NCH - SBB) // SUP
ACH = N_NODES // CHUNK
DEN_R = 80


def _tc_pre_body(x_ref, w_ref, ap_ref, msg_ref, al_ref):
    msg = jnp.dot(x_ref[...], w_ref[...], preferred_element_type=jnp.float32)
    msg_ref[...] = msg
    al_ref[...] = jnp.dot(msg, ap_ref[...], preferred_element_type=jnp.float32)


def _tc_pre(x, w, a_pad):
    blk = 1000
    grid = N_NODES // blk
    return pl.pallas_call(
        _tc_pre_body,
        grid=(grid,),
        in_specs=[
            pl.BlockSpec((blk, D), lambda i: (i, 0)),
            pl.BlockSpec((D, D), lambda i: (0, 0)),
            pl.BlockSpec((D, D), lambda i: (0, 0)),
        ],
        out_specs=[
            pl.BlockSpec((blk, D), lambda i: (i, 0)),
            pl.BlockSpec((blk, D), lambda i: (i, 0)),
        ],
        out_shape=[
            jax.ShapeDtypeStruct((N_NODES, D), jnp.float32),
            jax.ShapeDtypeStruct((N_NODES, D), jnp.float32),
        ],
    )(x, w, a_pad)


def _sc_a_body(row_h, col_h, nv_h, asrc_h, adst_h, w_h, den_h,
               rowa, cola, nva, wbuf, asrc_v, adst_v, denb):
    cid = lax.axis_index("c")
    sid = lax.axis_index("s")
    wid = cid * NS + sid

    pltpu.sync_copy(asrc_h, asrc_v)
    pltpu.sync_copy(adst_h, adst_v)

    z16 = jnp.zeros((16,), jnp.float32)

    def zden(r, _):
        for k in range(D // 16):
            denb[r, pl.ds(k * 16, 16)] = z16
        return 0

    lax.fori_loop(0, DEN_R, zden, 0)

    def stage(s, _):
        pltpu.sync_copy(row_h.at[wid, s], rowa)
        pltpu.sync_copy(col_h.at[wid, s], cola)
        pltpu.sync_copy(nv_h.at[wid, s], nva)

        def chunk(j, _):
            for g in range(CHUNK // 16):
                r16 = rowa[j, pl.ds(g * 16, 16)]
                c16 = cola[j, pl.ds(g * 16, 16)]
                logit = (plsc.load_gather(asrc_v, [r16]) +
                         plsc.load_gather(adst_v, [c16]))
                logit = jnp.where(logit >= 0, logit, NEG_SLOPE * logit)
                ex = jnp.exp(logit)
                wbuf[j, pl.ds(g * 16, 16)] = nva[j, pl.ds(g * 16, 16)] * ex
                hi = lax.shift_right_logical(r16, 7)
                lo = jnp.bitwise_and(r16, 127)
                plsc.addupdate_scatter(denb, [hi, lo], ex)
            return 0

        lax.fori_loop(0, ECA, chunk, 0)
        pltpu.sync_copy(wbuf, w_h.at[wid, s])
        return 0

    lax.fori_loop(0, NSA, stage, 0)
    pltpu.sync_copy(denb, den_h.at[cid, sid])


def _sc_a(row4d, col4d, nv4d, a_src, a_dst):
    mesh = plsc.VectorSubcoreMesh(core_axis_name="c", subcore_axis_name="s")
    f = pl.kernel(
        _sc_a_body,
        out_type=[
            jax.ShapeDtypeStruct((NW, NSA, ECA, CHUNK), jnp.float32),
            jax.ShapeDtypeStruct((NC, NS, DEN_R, D), jnp.float32),
        ],
        mesh=mesh,
        scratch_types=[
            pltpu.VMEM((ECA, CHUNK), jnp.int32),
            pltpu.VMEM((ECA, CHUNK), jnp.int32),
            pltpu.VMEM((ECA, CHUNK), jnp.float32),
            pltpu.VMEM((ECA, CHUNK), jnp.float32),
            pltpu.VMEM((N_NODES,), jnp.float32),
            pltpu.VMEM((N_NODES,), jnp.float32),
            pltpu.VMEM((DEN_R, D), jnp.float32),
        ],
        compiler_params=pltpu.CompilerParams(needs_layout_passes=False),
    )
    return f(row4d, col4d, nv4d, a_src, a_dst)


def _sc_b_body(row_h, col_h, w_h, msg_h, num_h,
               rows0, rows1, rows2,
               ecr0, ecr1, ecr2, ecc0, ecc1, ecc2, ecw0, ecw1, ecw2,
               w_v, acc_sh, g0, g1, g2, s0, s1, s2, esem):
    cid = lax.axis_index("c")
    sid = lax.axis_index("s")
    wid = cid * NS + sid
    bufs = [rows0, rows1, rows2]
    ecrs = [ecr0, ecr1, ecr2]
    eccs = [ecc0, ecc1, ecc2]
    ecws = [ecw0, ecw1, ecw2]
    gsems = [g0, g1, g2]
    ssems = [s0, s1, s2]

    z16 = jnp.zeros((16,), jnp.float32)

    def zr(r, _):
        for k in range(D // 16):
            rows0[r, pl.ds(k * 16, 16)] = z16
        return 0

    lax.fori_loop(0, CHUNK, zr, 0)
    for k in range(ACH // NS + 1):
        t = sid + NS * k

        @pl.when(t < ACH)
        def _():
            pltpu.sync_copy(rows0, acc_sh.at[pl.ds(t * CHUNK, CHUNK)])

    plsc.subcore_barrier()

    for sl in (0, 1):
        pltpu.sync_copy(row_h.at[wid, sl], ecrs[sl])
        pltpu.sync_copy(col_h.at[wid, sl], eccs[sl])
        pltpu.sync_copy(w_h.at[wid, sl], ecws[sl])

    pltpu.async_copy(msg_h.at[eccs[0].at[0]], bufs[0], gsems[0])
    pltpu.async_copy(msg_h.at[eccs[0].at[1]], bufs[1], gsems[1])

    def step(p, t):
        u = p * SUP + t
        b = t % 3
        sl = (t // SBB) % 3
        jj = t % SBB

        pltpu.make_async_copy(
            msg_h.at[pl.ds(0, CHUNK)], bufs[b], gsems[b]).wait()

        for g in range(CHUNK // 16):
            w_v[pl.ds(g * 16, 16)] = ecws[sl][jj, pl.ds(g * 16, 16)]

        def edge4(q, _):
            for dd in range(4):
                i = q * 4 + dd
                w = plsc.load_gather(w_v, [jnp.full((16,), i, jnp.int32)])
                for kk in range(D // 16):
                    bufs[b][i, pl.ds(kk * 16, 16)] = (
                        bufs[b][i, pl.ds(kk * 16, 16)] * w)
            return 0

        lax.fori_loop(0, CHUNK // 4, edge4, 0)

        pltpu.async_copy(bufs[b], acc_sh.at[ecrs[sl].at[jj]], ssems[b],
                         add=True)

        u2 = u + 2
        t2 = t + 2
        b2 = t2 % 3
        sl2 = (t2 // SBB) % 3
        jj2 = t2 % SBB

        @pl.when(u2 < NCH)
        def _():
            @pl.when(u >= 1)
            def _():
                pltpu.make_async_copy(
                    bufs[b2], acc_sh.at[pl.ds(0, CHUNK)], ssems[b2]).wait()

            if jj2 == 0:
                @pl.when(u2 >= 2 * SBB)
                def _():
                    pltpu.make_async_copy(
                        row_h.at[wid, 0], ecrs[sl2], esem).wait()
                    pltpu.make_async_copy(
                        col_h.at[wid, 0], eccs[sl2], esem).wait()
                    pltpu.make_async_copy(
                        w_h.at[wid, 0], ecws[sl2], esem).wait()

            if jj2 == 3:
                sn = lax.div(u2, SBB) + 2
                sln = (sl2 + 2) % 3

                @pl.when(sn < NST)
                def _():
                    pltpu.async_copy(row_h.at[wid, sn], ecrs[sln], esem)
                    pltpu.async_copy(col_h.at[wid, sn], eccs[sln], esem)
                    pltpu.async_copy(w_h.at[wid, sn], ecws[sln], esem)

            pltpu.async_copy(msg_h.at[eccs[sl2].at[jj2]], bufs[b2], gsems[b2])

    def outer(p, _):
        for t in range(SUP):
            step(p, t)
        return 0

    lax.fori_loop(0, NOUT, outer, 0)
    for t in range(SBB):
        step(jnp.int32(NOUT), t)

    for b in (0, 1, 2):
        pltpu.make_async_copy(
            bufs[b], acc_sh.at[pl.ds(0, CHUNK)], ssems[b]).wait()

    plsc.subcore_barrier()
    for k in range(ACH // NS + 1):
        t = sid + NS * k

        @pl.when(t < ACH)
        def _():
            pltpu.sync_copy(acc_sh.at[pl.ds(t * CHUNK, CHUNK)],
                            num_h.at[cid, pl.ds(t * CHUNK, CHUNK)])


def _sc_b(row4d, col4d, w4d, msg):
    mesh = plsc.VectorSubcoreMesh(core_axis_name="c", subcore_axis_name="s")
    f = pl.kernel(
        _sc_b_body,
        out_type=jax.ShapeDtypeStruct((NC, N_NODES, D), jnp.float32),
        mesh=mesh,
        scratch_types=[
            pltpu.VMEM((CHUNK, D), jnp.float32),
            pltpu.VMEM((CHUNK, D), jnp.float32),
            pltpu.VMEM((CHUNK, D), jnp.float32),
            pltpu.VMEM((SBB, CHUNK), jnp.int32),
            pltpu.VMEM((SBB, CHUNK), jnp.int32),
            pltpu.VMEM((SBB, CHUNK), jnp.int32),
            pltpu.VMEM((SBB, CHUNK), jnp.int32),
            pltpu.VMEM((SBB, CHUNK), jnp.int32),
            pltpu.VMEM((SBB, CHUNK), jnp.int32),
            pltpu.VMEM((SBB, CHUNK), jnp.float32),
            pltpu.VMEM((SBB, CHUNK), jnp.float32),
            pltpu.VMEM((SBB, CHUNK), jnp.float32),
            pltpu.VMEM((CHUNK,), jnp.float32),
            pltpu.VMEM_SHARED((N_NODES, D), jnp.float32),
            pltpu.SemaphoreType.DMA,
            pltpu.SemaphoreType.DMA,
            pltpu.SemaphoreType.DMA,
            pltpu.SemaphoreType.DMA,
            pltpu.SemaphoreType.DMA,
            pltpu.SemaphoreType.DMA,
            pltpu.SemaphoreType.DMA,
        ],
        compiler_params=pltpu.CompilerParams(needs_layout_passes=False),
    )
    return f(row4d, col4d, w4d, msg)


def _tc_fin_body(np_ref, dp_ref, out_ref):
    n0 = np_ref[0]
    n1 = np_ref[1]
    num = n0 + n1
    den = jnp.sum(dp_ref[...], axis=1)[:, None]
    safe = den > 0
    inv = jnp.where(safe, 1.0 / jnp.where(safe, den, 1.0), 0.0)
    out_ref[...] = num * inv


def _tc_fin(num_part, den_part):
    blk = 1000
    grid = N_NODES // blk
    return pl.pallas_call(
        _tc_fin_body,
        grid=(grid,),
        in_specs=[
            pl.BlockSpec((NC, blk, D), lambda i: (0, i, 0)),
            pl.BlockSpec((blk, NW), lambda i: (i, 0)),
        ],
        out_specs=pl.BlockSpec((blk, D), lambda i: (i, 0)),
        out_shape=jax.ShapeDtypeStruct((N_NODES, D), jnp.float32),
    )(num_part, den_part)


@jax.jit
def kernel(x, edge_index, neighborhood_values, W, a):
    row4d = edge_index[0].reshape(NW, NSA, ECA, CHUNK)
    col4d = edge_index[1].reshape(NW, NSA, ECA, CHUNK)
    nv4d = neighborhood_values.reshape(NW, NSA, ECA, CHUNK)
    a_pad = jnp.zeros((D, D), jnp.float32)
    a_pad = a_pad.at[:, 0].set(a[:D, 0]).at[:, 1].set(a[D:, 0])

    msg, alphas = _tc_pre(x, W, a_pad)
    a_src = alphas[:, 0]
    a_dst = alphas[:, 1]

    w4d, den_part = _sc_a(row4d, col4d, nv4d, a_src, a_dst)
    row4b = edge_index[0].reshape(NW, NST, SBB, CHUNK)
    col4b = edge_index[1].reshape(NW, NST, SBB, CHUNK)
    w4b = w4d.reshape(NW, NST, SBB, CHUNK)
    num_part = _sc_b(row4b, col4b, w4b, msg)

    den2d = den_part.reshape(NW, DEN_R * D)[:, :N_NODES].T
    return _tc_fin(num_part, den2d)
